# Initial kernel scaffold; baseline (speedup 1.0000x reference)
import jax
import jax.numpy as jnp
from jax.experimental import pallas as pl


def kernel(x, edge_index, batch, params):
    raise NotImplementedError("write your pallas kernel here")


# speedup vs baseline: 77.3839x; 77.3839x over previous
import functools
import jax
import jax.numpy as jnp
from jax import lax
from jax.experimental import pallas as pl
from jax.experimental.pallas import tpu as pltpu
from jax.experimental.pallas import tpu_sc as plsc

NC = 2
NS = 16
NW = NC * NS
CK = 128
NEG = 0.2

_MESH = dict(core_axis_name="c", subcore_axis_name="s", num_cores=NC,
             num_subcores=NS)


def _tc1_body(x_ref, w_ref, as_ref, ad_ref, h_ref, s_ref, d_ref, m_ref,
              ms_acc, md_acc):
    i = pl.program_id(0)
    h = jnp.dot(x_ref[...], w_ref[...], preferred_element_type=jnp.float32)
    h_ref[...] = h
    s = jnp.dot(h, as_ref[...], preferred_element_type=jnp.float32)
    d = jnp.dot(h, ad_ref[...], preferred_element_type=jnp.float32)
    s_ref[...] = s
    d_ref[...] = d

    @pl.when(i == 0)
    def _():
        ms_acc[...] = jnp.full((8, 16), -1e30, jnp.float32)
        md_acc[...] = jnp.full((8, 16), -1e30, jnp.float32)

    ms_acc[...] = jnp.maximum(
        ms_acc[...], jnp.broadcast_to(jnp.max(s, 0, keepdims=True), (8, 16)))
    md_acc[...] = jnp.maximum(
        md_acc[...], jnp.broadcast_to(jnp.max(d, 0, keepdims=True), (8, 16)))

    @pl.when(i == pl.num_programs(0) - 1)
    def _():
        m_ref[...] = jnp.maximum(ms_acc[...] + md_acc[...], 0.0)


def _tc1(x, w1s, a_s, a_d, n, bn):
    g = n // bn
    return pl.pallas_call(
        _tc1_body,
        grid=(g,),
        in_specs=[
            pl.BlockSpec((bn, 128), lambda i: (i, 0)),
            pl.BlockSpec((128, 128), lambda i: (0, 0)),
            pl.BlockSpec((128, 16), lambda i: (0, 0)),
            pl.BlockSpec((128, 16), lambda i: (0, 0)),
        ],
        out_specs=[
            pl.BlockSpec((bn, 128), lambda i: (i, 0)),
            pl.BlockSpec((bn, 16), lambda i: (i, 0)),
            pl.BlockSpec((bn, 16), lambda i: (i, 0)),
            pl.BlockSpec((8, 16), lambda i: (0, 0)),
        ],
        out_shape=[
            jax.ShapeDtypeStruct((n, 128), jnp.float32),
            jax.ShapeDtypeStruct((n, 16), jnp.float32),
            jax.ShapeDtypeStruct((n, 16), jnp.float32),
            jax.ShapeDtypeStruct((8, 16), jnp.float32),
        ],
        scratch_shapes=[pltpu.VMEM((8, 16), jnp.float32),
                        pltpu.VMEM((8, 16), jnp.float32)],
    )(x, w1s, a_s, a_d)


def _tc2_body(aggp_ref, denp_ref, b1_ref, e16_ref, w2_ref, a2s_ref, a2d_ref,
              h2_ref, s_ref, d_ref, m_ref, ms_acc, md_acc):
    i = pl.program_id(0)
    agg = aggp_ref[0] + aggp_ref[1]
    den = denp_ref[0] + denp_ref[1]
    dinv = 1.0 / (den + 1e-16)
    dexp = jnp.dot(dinv, e16_ref[...], preferred_element_type=jnp.float32)
    h1 = jnp.maximum(agg * dexp + b1_ref[0:1, :], 0.0)
    h2 = jnp.dot(h1, w2_ref[...], preferred_element_type=jnp.float32)
    h2_ref[...] = h2
    s = jnp.dot(h2, a2s_ref[...], preferred_element_type=jnp.float32)
    d = jnp.dot(h2, a2d_ref[...], preferred_element_type=jnp.float32)
    s_ref[...] = s
    d_ref[...] = d

    @pl.when(i == 0)
    def _():
        ms_acc[...] = jnp.full((8, 16), -1e30, jnp.float32)
        md_acc[...] = jnp.full((8, 16), -1e30, jnp.float32)

    ms_acc[...] = jnp.maximum(
        ms_acc[...], jnp.broadcast_to(jnp.max(s, 0, keepdims=True), (8, 16)))
    md_acc[...] = jnp.maximum(
        md_acc[...], jnp.broadcast_to(jnp.max(d, 0, keepdims=True), (8, 16)))

    @pl.when(i == pl.num_programs(0) - 1)
    def _():
        m_ref[...] = jnp.maximum(ms_acc[...] + md_acc[...], 0.0)


def _tc2(aggp, denp, b1r, e16, w2s, a2s, a2d, n, bn):
    g = n // bn
    return pl.pallas_call(
        _tc2_body,
        grid=(g,),
        in_specs=[
            pl.BlockSpec((2, bn, 128), lambda i: (0, i, 0)),
            pl.BlockSpec((2, bn, 16), lambda i: (0, i, 0)),
            pl.BlockSpec((8, 128), lambda i: (0, 0)),
            pl.BlockSpec((16, 128), lambda i: (0, 0)),
            pl.BlockSpec((128, 1024), lambda i: (0, 0)),
            pl.BlockSpec((1024, 16), lambda i: (0, 0)),
            pl.BlockSpec((1024, 16), lambda i: (0, 0)),
        ],
        out_specs=[
            pl.BlockSpec((bn, 1024), lambda i: (i, 0)),
            pl.BlockSpec((bn, 16), lambda i: (i, 0)),
            pl.BlockSpec((bn, 16), lambda i: (i, 0)),
            pl.BlockSpec((8, 16), lambda i: (0, 0)),
        ],
        out_shape=[
            jax.ShapeDtypeStruct((n, 1024), jnp.float32),
            jax.ShapeDtypeStruct((n, 16), jnp.float32),
            jax.ShapeDtypeStruct((n, 16), jnp.float32),
            jax.ShapeDtypeStruct((8, 16), jnp.float32),
        ],
        scratch_shapes=[pltpu.VMEM((8, 16), jnp.float32),
                        pltpu.VMEM((8, 16), jnp.float32)],
    )(aggp, denp, b1r, e16, w2s, a2s, a2d)


def _tc3_body(denp_ref, out_ref):
    out_ref[...] = 1.0 / (denp_ref[0] + denp_ref[1] + 1e-16)


def _tc3(den2p, np_):
    return pl.pallas_call(
        _tc3_body,
        out_shape=jax.ShapeDtypeStruct((np_, 16), jnp.float32),
    )(den2p)


def _tc4_body(wp_ref, h2_ref, e2_ref, wcb_ref, cv_ref, out_ref, t_acc,
              *, scale):
    i = pl.program_id(0)
    w = wp_ref[0] + wp_ref[1]
    wexp = jnp.dot(w, e2_ref[...], preferred_element_type=jnp.float32)
    t = jnp.sum(wexp * h2_ref[...], axis=0, keepdims=True)

    @pl.when(i == 0)
    def _():
        t_acc[...] = jnp.zeros((8, 1024), jnp.float32)

    t_acc[...] = t_acc[...] + jnp.broadcast_to(t, (8, 1024))

    @pl.when(i == pl.num_programs(0) - 1)
    def _():
        out_ref[...] = (
            jnp.dot(t_acc[...], wcb_ref[...],
                    preferred_element_type=jnp.float32) * scale
            + cv_ref[...])


def _tc4(wp, h2, e2, wcb, cv, n, bn):
    g = n // bn
    return pl.pallas_call(
        functools.partial(_tc4_body, scale=1.0 / (16.0 * n)),
        grid=(g,),
        in_specs=[
            pl.BlockSpec((2, bn, 16), lambda i: (0, i, 0)),
            pl.BlockSpec((bn, 1024), lambda i: (i, 0)),
            pl.BlockSpec((16, 1024), lambda i: (0, 0)),
            pl.BlockSpec((1024, 8), lambda i: (0, 0)),
            pl.BlockSpec((8, 8), lambda i: (0, 0)),
        ],
        out_specs=[pl.BlockSpec((8, 8), lambda i: (0, 0))],
        out_shape=[jax.ShapeDtypeStruct((8, 8), jnp.float32)],
        scratch_shapes=[pltpu.VMEM((8, 1024), jnp.float32)],
    )(wp, h2, e2, wcb, cv)[0]


def _sc_l1_body(src_h, dst_h, asrc_h, adst_h, h1_h, m_h, z16_h, z128_h,
                den_out, agg_out,
                sidx, didx, abuf, bbuf, eebuf, hbuf, sbuf, mbuf,
                den_sh, agg_sh, sem, *, np_, chunks):
    c = lax.axis_index("c")
    s = lax.axis_index("s")
    wid = c * NS + s
    rows = np_ // NS
    pltpu.sync_copy(z16_h, den_sh.at[pl.ds(s * rows, rows)])
    pltpu.sync_copy(z128_h, agg_sh.at[pl.ds(s * rows, rows)])
    pltpu.sync_copy(m_h.at[0], mbuf)
    plsc.subcore_barrier()

    base = wid * (chunks * CK)
    lane = lax.iota(jnp.int32, 16)

    def chunk(ci, carry):
        off = base + ci * CK
        pltpu.sync_copy(src_h.at[pl.ds(off, CK)], sidx)
        pltpu.sync_copy(dst_h.at[pl.ds(off, CK)], didx)
        pltpu.async_copy(asrc_h.at[sidx], abuf, sem).wait()
        pltpu.async_copy(adst_h.at[didx], bbuf, sem).wait()
        pltpu.async_copy(h1_h.at[sidx], hbuf, sem).wait()
        m = mbuf[...]

        def row(i, carry2):
            v = abuf[i] + bbuf[i]
            v = jnp.where(v >= 0.0, v, v * NEG)
            eebuf[i] = jnp.exp(v - m)
            return carry2

        lax.fori_loop(0, CK, row, 0, unroll=2)

        def row2(i, carry2):
            er = eebuf[i]
            for j in range(8):
                colj = (lane + 16 * j) >> 3
                sc = lax.gather(
                    er, colj[:, None],
                    dimension_numbers=lax.GatherDimensionNumbers(
                        offset_dims=(), collapsed_slice_dims=(0,),
                        start_index_map=(0,)),
                    slice_sizes=(1,),
                    mode=lax.GatherScatterMode.PROMISE_IN_BOUNDS)
                sbuf[i, pl.ds(16 * j, 16)] = hbuf[i, pl.ds(16 * j, 16)] * sc
            return carry2

        lax.fori_loop(0, CK, row2, 0)

        pltpu.sync_copy(eebuf, den_sh.at[didx], add=True)
        pltpu.sync_copy(sbuf, agg_sh.at[didx], add=True)
        return carry

    lax.fori_loop(0, chunks, chunk, 0)
    plsc.subcore_barrier()
    pltpu.sync_copy(den_sh.at[pl.ds(s * rows, rows)],
                    den_out.at[c, pl.ds(s * rows, rows)])
    pltpu.sync_copy(agg_sh.at[pl.ds(s * rows, rows)],
                    agg_out.at[c, pl.ds(s * rows, rows)])


def _sc_l1(srcp, dstp, asrcp, adstp, h1p, m1, z16, z128, np_, chunks):
    body = functools.partial(_sc_l1_body, np_=np_, chunks=chunks)
    return pl.kernel(
        body,
        out_type=(jax.ShapeDtypeStruct((NC, np_, 16), jnp.float32),
                  jax.ShapeDtypeStruct((NC, np_, 128), jnp.float32)),
        mesh=plsc.VectorSubcoreMesh(**_MESH),
        compiler_params=pltpu.CompilerParams(use_tc_tiling_on_sc=False),
        scratch_types=(
            pltpu.VMEM((CK,), jnp.int32),
            pltpu.VMEM((CK,), jnp.int32),
            pltpu.VMEM((CK, 16), jnp.float32),
            pltpu.VMEM((CK, 16), jnp.float32),
            pltpu.VMEM((CK, 16), jnp.float32),
            pltpu.VMEM((CK, 128), jnp.float32),
            pltpu.VMEM((CK, 128), jnp.float32),
            pltpu.VMEM((16,), jnp.float32),
            pltpu.VMEM_SHARED((np_, 16), jnp.float32),
            pltpu.VMEM_SHARED((np_, 128), jnp.float32),
            pltpu.SemaphoreType.DMA,
        ),
    )(srcp, dstp, asrcp, adstp, h1p, m1, z16, z128)


def _sc_l2a_body(src_h, dst_h, asrc_h, adst_h, m_h, z16_h,
                 ee_out, den_out,
                 sidx, didx, abuf, bbuf, eebuf, mbuf, den_sh, sem,
                 *, np_, chunks):
    c = lax.axis_index("c")
    s = lax.axis_index("s")
    wid = c * NS + s
    rows = np_ // NS
    pltpu.sync_copy(z16_h, den_sh.at[pl.ds(s * rows, rows)])
    pltpu.sync_copy(m_h.at[0], mbuf)
    plsc.subcore_barrier()

    base = wid * (chunks * CK)

    def chunk(ci, carry):
        off = base + ci * CK
        pltpu.sync_copy(src_h.at[pl.ds(off, CK)], sidx)
        pltpu.sync_copy(dst_h.at[pl.ds(off, CK)], didx)
        pltpu.async_copy(asrc_h.at[sidx], abuf, sem).wait()
        pltpu.async_copy(adst_h.at[didx], bbuf, sem).wait()
        m = mbuf[...]

        def row(i, carry2):
            v = abuf[i] + bbuf[i]
            v = jnp.where(v >= 0.0, v, v * NEG)
            eebuf[i] = jnp.exp(v - m)
            return carry2

        lax.fori_loop(0, CK, row, 0, unroll=2)

        pltpu.sync_copy(eebuf, ee_out.at[pl.ds(off, CK)])
        pltpu.sync_copy(eebuf, den_sh.at[didx], add=True)
        return carry

    lax.fori_loop(0, chunks, chunk, 0)
    plsc.subcore_barrier()
    pltpu.sync_copy(den_sh.at[pl.ds(s * rows, rows)],
                    den_out.at[c, pl.ds(s * rows, rows)])


def _sc_l2a(srcp, dstp, asrc2p, adst2p, m2, z16, np_, ep, chunks):
    body = functools.partial(_sc_l2a_body, np_=np_, chunks=chunks)
    return pl.kernel(
        body,
        out_type=(jax.ShapeDtypeStruct((ep, 16), jnp.float32),
                  jax.ShapeDtypeStruct((NC, np_, 16), jnp.float32)),
        mesh=plsc.VectorSubcoreMesh(**_MESH),
        compiler_params=pltpu.CompilerParams(use_tc_tiling_on_sc=False),
        scratch_types=(
            pltpu.VMEM((CK,), jnp.int32),
            pltpu.VMEM((CK,), jnp.int32),
            pltpu.VMEM((CK, 16), jnp.float32),
            pltpu.VMEM((CK, 16), jnp.float32),
            pltpu.VMEM((CK, 16), jnp.float32),
            pltpu.VMEM((16,), jnp.float32),
            pltpu.VMEM_SHARED((np_, 16), jnp.float32),
            pltpu.SemaphoreType.DMA,
        ),
    )(srcp, dstp, asrc2p, adst2p, m2, z16)


def _sc_l2b_body(src_h, dst_h, ee_h, dinv_h, z16_h,
                 w_out,
                 sidx, didx, eebuf, bbuf, wbuf, w_sh, sem,
                 *, np_, chunks):
    c = lax.axis_index("c")
    s = lax.axis_index("s")
    wid = c * NS + s
    rows = np_ // NS
    pltpu.sync_copy(z16_h, w_sh.at[pl.ds(s * rows, rows)])
    plsc.subcore_barrier()

    base = wid * (chunks * CK)

    def chunk(ci, carry):
        off = base + ci * CK
        pltpu.sync_copy(src_h.at[pl.ds(off, CK)], sidx)
        pltpu.sync_copy(dst_h.at[pl.ds(off, CK)], didx)
        pltpu.sync_copy(ee_h.at[pl.ds(off, CK)], eebuf)
        pltpu.async_copy(dinv_h.at[didx], bbuf, sem).wait()

        def row(i, carry2):
            wbuf[i] = eebuf[i] * bbuf[i]
            return carry2

        lax.fori_loop(0, CK, row, 0, unroll=2)

        pltpu.sync_copy(wbuf, w_sh.at[sidx], add=True)
        return carry

    lax.fori_loop(0, chunks, chunk, 0)
    plsc.subcore_barrier()
    pltpu.sync_copy(w_sh.at[pl.ds(s * rows, rows)],
                    w_out.at[c, pl.ds(s * rows, rows)])


def _sc_l2b(srcp, dstp, ee2, dinv, z16, np_, chunks):
    body = functools.partial(_sc_l2b_body, np_=np_, chunks=chunks)
    return pl.kernel(
        body,
        out_type=jax.ShapeDtypeStruct((NC, np_, 16), jnp.float32),
        mesh=plsc.VectorSubcoreMesh(**_MESH),
        compiler_params=pltpu.CompilerParams(use_tc_tiling_on_sc=False),
        scratch_types=(
            pltpu.VMEM((CK,), jnp.int32),
            pltpu.VMEM((CK,), jnp.int32),
            pltpu.VMEM((CK, 16), jnp.float32),
            pltpu.VMEM((CK, 16), jnp.float32),
            pltpu.VMEM((CK, 16), jnp.float32),
            pltpu.VMEM_SHARED((np_, 16), jnp.float32),
            pltpu.SemaphoreType.DMA,
        ),
    )(srcp, dstp, ee2, dinv, z16)


def _stack_weights(params):
    p0, p1 = params
    w1s = jnp.concatenate([p0['W1'], p1['W1']], axis=1)
    f_s = jnp.concatenate([p0['as1'].reshape(-1), p1['as1'].reshape(-1)])
    f_d = jnp.concatenate([p0['ad1'].reshape(-1), p1['ad1'].reshape(-1)])
    oh = jax.nn.one_hot(jnp.arange(128) // 8, 16, dtype=jnp.float32)
    a_s = oh * f_s[:, None]
    a_d = oh * f_d[:, None]
    b1r = jnp.broadcast_to(
        jnp.concatenate([p0['b1'], p1['b1']])[None, :], (8, 128))
    e16 = oh.T
    w2s = jnp.zeros((128, 1024), jnp.float32)
    w2s = w2s.at[:64, :512].set(p0['W2'])
    w2s = w2s.at[64:, 512:].set(p1['W2'])
    f2s = jnp.concatenate([p0['as2'].reshape(-1), p1['as2'].reshape(-1)])
    f2d = jnp.concatenate([p0['ad2'].reshape(-1), p1['ad2'].reshape(-1)])
    oh2 = jax.nn.one_hot(jnp.arange(1024) // 64, 16, dtype=jnp.float32)
    a2s = oh2 * f2s[:, None]
    a2d = oh2 * f2d[:, None]
    e2 = oh2.T
    wcb = jnp.concatenate([jnp.tile(p0['Wc'], (8, 1)),
                           jnp.tile(p1['Wc'], (8, 1))], axis=0)
    cv = 0.5 * ((p0['b2'] @ p0['Wc'] + p0['bc'])
                + (p1['b2'] @ p1['Wc'] + p1['bc']))
    cv = jnp.broadcast_to(cv[None, :], (8, 8))
    return w1s, a_s, a_d, b1r, e16, w2s, a2s, a2d, e2, wcb, cv


@jax.jit
def kernel(x, edge_index, batch, params):
    n = x.shape[0]
    e = edge_index.shape[1]
    bn = 1000
    np_ = -(-(n + 1) // 128) * 128
    chunks = -(-e // (NW * CK))
    ep = NW * CK * chunks
    src = edge_index[0]
    dst = edge_index[1]
    pad = jnp.full((ep - e,), n, jnp.int32)
    srcp = jnp.concatenate([src, pad])
    dstp = jnp.concatenate([dst, pad])
    rows = np_ // NS
    z16 = jnp.zeros((rows, 16), jnp.float32)
    z128 = jnp.zeros((rows, 128), jnp.float32)

    (w1s, a_s, a_d, b1r, e16, w2s, a2s, a2d, e2, wcb, cv) = \
        _stack_weights(params)

    h1, asrc, adst, m1 = _tc1(x, w1s, a_s, a_d, n, bn)
    h1p = jnp.pad(h1, ((0, np_ - n), (0, 0)))
    asrcp = jnp.pad(asrc, ((0, np_ - n), (0, 0)))
    adstp = jnp.pad(adst, ((0, np_ - n), (0, 0)))

    denp, aggp = _sc_l1(srcp, dstp, asrcp, adstp, h1p, m1, z16, z128,
                        np_, chunks)

    h2, asrc2, adst2, m2 = _tc2(aggp[:, :n], denp[:, :n], b1r, e16,
                                w2s, a2s, a2d, n, bn)
    asrc2p = jnp.pad(asrc2, ((0, np_ - n), (0, 0)))
    adst2p = jnp.pad(adst2, ((0, np_ - n), (0, 0)))

    ee2, den2p = _sc_l2a(srcp, dstp, asrc2p, adst2p, m2, z16,
                         np_, ep, chunks)
    dinv = _tc3(den2p, np_)
    wp = _sc_l2b(srcp, dstp, ee2, dinv, z16, np_, chunks)

    out = _tc4(wp[:, :n], h2, e2, wcb, cv, n, bn)
    return out[0:1]


# speedup vs baseline: 80.5298x; 1.0407x over previous
import functools
import jax
import jax.numpy as jnp
from jax import lax
from jax.experimental import pallas as pl
from jax.experimental.pallas import tpu as pltpu
from jax.experimental.pallas import tpu_sc as plsc

NC = 2
NS = 16
NW = NC * NS
CK = 128
NEG = 0.2

_MESH = dict(core_axis_name="c", subcore_axis_name="s", num_cores=NC,
             num_subcores=NS)
_SC_PARAMS = dict(
    compiler_params=pltpu.CompilerParams(use_tc_tiling_on_sc=False))


def _tc1_body(x_ref, w_ref, as_ref, ad_ref, comb_ref, d_ref, m_ref,
              ms_acc, md_acc):
    i = pl.program_id(0)
    h = jnp.dot(x_ref[...], w_ref[...], preferred_element_type=jnp.float32)
    s = jnp.dot(h, as_ref[...], preferred_element_type=jnp.float32)
    d = jnp.dot(h, ad_ref[...], preferred_element_type=jnp.float32)
    comb_ref[:, :128] = h
    comb_ref[:, 128:144] = s
    d_ref[...] = d

    @pl.when(i == 0)
    def _():
        ms_acc[...] = jnp.full((8, 16), -1e30, jnp.float32)
        md_acc[...] = jnp.full((8, 16), -1e30, jnp.float32)

    ms_acc[...] = jnp.maximum(
        ms_acc[...], jnp.broadcast_to(jnp.max(s, 0, keepdims=True), (8, 16)))
    md_acc[...] = jnp.maximum(
        md_acc[...], jnp.broadcast_to(jnp.max(d, 0, keepdims=True), (8, 16)))

    @pl.when(i == pl.num_programs(0) - 1)
    def _():
        m_ref[...] = jnp.maximum(ms_acc[...] + md_acc[...], 0.0)


def _tc1(x, w1s, a_s, a_d, n, np_, bn):
    g = n // bn
    return pl.pallas_call(
        _tc1_body,
        grid=(g,),
        in_specs=[
            pl.BlockSpec((bn, 128), lambda i: (i, 0)),
            pl.BlockSpec((128, 128), lambda i: (0, 0)),
            pl.BlockSpec((128, 16), lambda i: (0, 0)),
            pl.BlockSpec((128, 16), lambda i: (0, 0)),
        ],
        out_specs=[
            pl.BlockSpec((bn, 144), lambda i: (i, 0)),
            pl.BlockSpec((bn, 16), lambda i: (i, 0)),
            pl.BlockSpec((8, 16), lambda i: (0, 0)),
        ],
        out_shape=[
            jax.ShapeDtypeStruct((np_, 144), jnp.float32),
            jax.ShapeDtypeStruct((np_, 16), jnp.float32),
            jax.ShapeDtypeStruct((8, 16), jnp.float32),
        ],
        scratch_shapes=[pltpu.VMEM((8, 16), jnp.float32),
                        pltpu.VMEM((8, 16), jnp.float32)],
    )(x, w1s, a_s, a_d)


def _tc2_body(accp_ref, b1_ref, e16_ref, w2_ref, a2s_ref, a2d_ref,
              h2_ref, s_ref, d_ref, m_ref, ms_acc, md_acc):
    i = pl.program_id(0)
    comb = accp_ref[0] + accp_ref[1]
    agg = comb[:, :128]
    den = comb[:, 128:144]
    dinv = 1.0 / (den + 1e-16)
    dexp = jnp.dot(dinv, e16_ref[...], preferred_element_type=jnp.float32)
    h1 = jnp.maximum(agg * dexp + b1_ref[0:1, :], 0.0)
    h2 = jnp.dot(h1, w2_ref[...], preferred_element_type=jnp.float32)
    h2_ref[...] = h2
    s = jnp.dot(h2, a2s_ref[...], preferred_element_type=jnp.float32)
    d = jnp.dot(h2, a2d_ref[...], preferred_element_type=jnp.float32)
    s_ref[...] = s
    d_ref[...] = d

    @pl.when(i == 0)
    def _():
        ms_acc[...] = jnp.full((8, 16), -1e30, jnp.float32)
        md_acc[...] = jnp.full((8, 16), -1e30, jnp.float32)

    ms_acc[...] = jnp.maximum(
        ms_acc[...], jnp.broadcast_to(jnp.max(s, 0, keepdims=True), (8, 16)))
    md_acc[...] = jnp.maximum(
        md_acc[...], jnp.broadcast_to(jnp.max(d, 0, keepdims=True), (8, 16)))

    @pl.when(i == pl.num_programs(0) - 1)
    def _():
        m_ref[...] = jnp.maximum(ms_acc[...] + md_acc[...], 0.0)


def _tc2(accp, b1r, e16, w2s, a2s, a2d, n, np_, bn):
    g = n // bn
    return pl.pallas_call(
        _tc2_body,
        grid=(g,),
        in_specs=[
            pl.BlockSpec((2, bn, 144), lambda i: (0, i, 0)),
            pl.BlockSpec((8, 128), lambda i: (0, 0)),
            pl.BlockSpec((16, 128), lambda i: (0, 0)),
            pl.BlockSpec((128, 1024), lambda i: (0, 0)),
            pl.BlockSpec((1024, 16), lambda i: (0, 0)),
            pl.BlockSpec((1024, 16), lambda i: (0, 0)),
        ],
        out_specs=[
            pl.BlockSpec((bn, 1024), lambda i: (i, 0)),
            pl.BlockSpec((bn, 16), lambda i: (i, 0)),
            pl.BlockSpec((bn, 16), lambda i: (i, 0)),
            pl.BlockSpec((8, 16), lambda i: (0, 0)),
        ],
        out_shape=[
            jax.ShapeDtypeStruct((n, 1024), jnp.float32),
            jax.ShapeDtypeStruct((np_, 16), jnp.float32),
            jax.ShapeDtypeStruct((np_, 16), jnp.float32),
            jax.ShapeDtypeStruct((8, 16), jnp.float32),
        ],
        scratch_shapes=[pltpu.VMEM((8, 16), jnp.float32),
                        pltpu.VMEM((8, 16), jnp.float32)],
    )(accp, b1r, e16, w2s, a2s, a2d)


def _tc3_body(denp_ref, out_ref):
    out_ref[...] = 1.0 / (denp_ref[0] + denp_ref[1] + 1e-16)


def _tc3(den2p, np_):
    return pl.pallas_call(
        _tc3_body,
        out_shape=jax.ShapeDtypeStruct((np_, 16), jnp.float32),
    )(den2p)


def _tc4_body(wp_ref, h2_ref, e2_ref, wcb_ref, cv_ref, out_ref, t_acc,
              *, scale):
    i = pl.program_id(0)
    w = wp_ref[0] + wp_ref[1]
    wexp = jnp.dot(w, e2_ref[...], preferred_element_type=jnp.float32)
    t = jnp.sum(wexp * h2_ref[...], axis=0, keepdims=True)

    @pl.when(i == 0)
    def _():
        t_acc[...] = jnp.zeros((8, 1024), jnp.float32)

    t_acc[...] = t_acc[...] + jnp.broadcast_to(t, (8, 1024))

    @pl.when(i == pl.num_programs(0) - 1)
    def _():
        out_ref[...] = (
            jnp.dot(t_acc[...], wcb_ref[...],
                    preferred_element_type=jnp.float32) * scale
            + cv_ref[...])


def _tc4(wp, h2, e2, wcb, cv, n, bn):
    g = n // bn
    return pl.pallas_call(
        functools.partial(_tc4_body, scale=1.0 / (16.0 * n)),
        grid=(g,),
        in_specs=[
            pl.BlockSpec((2, bn, 16), lambda i: (0, i, 0)),
            pl.BlockSpec((bn, 1024), lambda i: (i, 0)),
            pl.BlockSpec((16, 1024), lambda i: (0, 0)),
            pl.BlockSpec((1024, 8), lambda i: (0, 0)),
            pl.BlockSpec((8, 8), lambda i: (0, 0)),
        ],
        out_specs=[pl.BlockSpec((8, 8), lambda i: (0, 0))],
        out_shape=[jax.ShapeDtypeStruct((8, 8), jnp.float32)],
        scratch_shapes=[pltpu.VMEM((8, 1024), jnp.float32)],
    )(wp, h2, e2, wcb, cv)[0]


def _sc_l1_body(src_h, dst_h, comb_h, adst_h, m_h, z144_h,
                acc_out,
                sidx, didx, gbuf, bbuf, sbuf, mbuf,
                acc_sh, sem, sem2, *, np_, chunks):
    c = lax.axis_index("c")
    s = lax.axis_index("s")
    wid = c * NS + s
    rows = np_ // NS
    pltpu.sync_copy(z144_h, acc_sh.at[pl.ds(s * rows, rows)])
    pltpu.sync_copy(m_h.at[0], mbuf)
    plsc.subcore_barrier()

    base = wid * (chunks * CK)
    lane = lax.iota(jnp.int32, 16)
    dnums = lax.GatherDimensionNumbers(
        offset_dims=(), collapsed_slice_dims=(0,), start_index_map=(0,))

    def chunk(ci, carry):
        off = base + ci * CK
        i1 = pltpu.async_copy(src_h.at[pl.ds(off, CK)], sidx, sem2)
        i2 = pltpu.async_copy(dst_h.at[pl.ds(off, CK)], didx, sem2)
        i1.wait()
        i2.wait()
        g1 = pltpu.async_copy(comb_h.at[sidx], gbuf, sem)
        g2 = pltpu.async_copy(adst_h.at[didx], bbuf, sem)
        g1.wait()
        g2.wait()
        m = mbuf[...]

        def row(i, carry2):
            v = gbuf[i, pl.ds(128, 16)] + bbuf[i]
            v = jnp.where(v >= 0.0, v, v * NEG)
            ee = jnp.exp(v - m)
            sbuf[i, pl.ds(128, 16)] = ee
            for j in range(8):
                colj = (lane + 16 * j) >> 3
                sc = lax.gather(
                    ee, colj[:, None], dimension_numbers=dnums,
                    slice_sizes=(1,),
                    mode=lax.GatherScatterMode.PROMISE_IN_BOUNDS)
                sbuf[i, pl.ds(16 * j, 16)] = gbuf[i, pl.ds(16 * j, 16)] * sc
            return carry2

        lax.fori_loop(0, CK, row, 0, unroll=2)

        pltpu.sync_copy(sbuf, acc_sh.at[didx], add=True)
        return carry

    lax.fori_loop(0, chunks, chunk, 0)
    plsc.subcore_barrier()
    pltpu.sync_copy(acc_sh.at[pl.ds(s * rows, rows)],
                    acc_out.at[c, pl.ds(s * rows, rows)])


def _sc_l1(srcp, dstp, comb, adst, m1, z144, np_, chunks):
    body = functools.partial(_sc_l1_body, np_=np_, chunks=chunks)
    return pl.kernel(
        body,
        out_type=jax.ShapeDtypeStruct((NC, np_, 144), jnp.float32),
        mesh=plsc.VectorSubcoreMesh(**_MESH),
        scratch_types=(
            pltpu.VMEM((CK,), jnp.int32),
            pltpu.VMEM((CK,), jnp.int32),
            pltpu.VMEM((CK, 144), jnp.float32),
            pltpu.VMEM((CK, 16), jnp.float32),
            pltpu.VMEM((CK, 144), jnp.float32),
            pltpu.VMEM((16,), jnp.float32),
            pltpu.VMEM_SHARED((np_, 144), jnp.float32),
            pltpu.SemaphoreType.DMA,
            pltpu.SemaphoreType.DMA,
        ),
        **_SC_PARAMS,
    )(srcp, dstp, comb, adst, m1, z144)


def _sc_l2a_body(src_h, dst_h, asrc_h, adst_h, m_h, z16_h,
                 ee_out, den_out,
                 sidx, didx, abuf, bbuf, eebuf, mbuf, den_sh, sem, sem2,
                 *, np_, chunks):
    c = lax.axis_index("c")
    s = lax.axis_index("s")
    wid = c * NS + s
    rows = np_ // NS
    pltpu.sync_copy(z16_h, den_sh.at[pl.ds(s * rows, rows)])
    pltpu.sync_copy(m_h.at[0], mbuf)
    plsc.subcore_barrier()

    base = wid * (chunks * CK)

    def chunk(ci, carry):
        off = base + ci * CK
        i1 = pltpu.async_copy(src_h.at[pl.ds(off, CK)], sidx, sem2)
        i2 = pltpu.async_copy(dst_h.at[pl.ds(off, CK)], didx, sem2)
        i1.wait()
        i2.wait()
        g1 = pltpu.async_copy(asrc_h.at[sidx], abuf, sem)
        g2 = pltpu.async_copy(adst_h.at[didx], bbuf, sem)
        g1.wait()
        g2.wait()
        m = mbuf[...]

        def row(i, carry2):
            v = abuf[i] + bbuf[i]
            v = jnp.where(v >= 0.0, v, v * NEG)
            eebuf[i] = jnp.exp(v - m)
            return carry2

        lax.fori_loop(0, CK, row, 0, unroll=4)

        w1 = pltpu.async_copy(eebuf, ee_out.at[pl.ds(off, CK)], sem2)
        pltpu.sync_copy(eebuf, den_sh.at[didx], add=True)
        w1.wait()
        return carry

    lax.fori_loop(0, chunks, chunk, 0)
    plsc.subcore_barrier()
    pltpu.sync_copy(den_sh.at[pl.ds(s * rows, rows)],
                    den_out.at[c, pl.ds(s * rows, rows)])


def _sc_l2a(srcp, dstp, asrc2, adst2, m2, z16, np_, ep, chunks):
    body = functools.partial(_sc_l2a_body, np_=np_, chunks=chunks)
    return pl.kernel(
        body,
        out_type=(jax.ShapeDtypeStruct((ep, 16), jnp.float32),
                  jax.ShapeDtypeStruct((NC, np_, 16), jnp.float32)),
        mesh=plsc.VectorSubcoreMesh(**_MESH),
        scratch_types=(
            pltpu.VMEM((CK,), jnp.int32),
            pltpu.VMEM((CK,), jnp.int32),
            pltpu.VMEM((CK, 16), jnp.float32),
            pltpu.VMEM((CK, 16), jnp.float32),
            pltpu.VMEM((CK, 16), jnp.float32),
            pltpu.VMEM((16,), jnp.float32),
            pltpu.VMEM_SHARED((np_, 16), jnp.float32),
            pltpu.SemaphoreType.DMA,
            pltpu.SemaphoreType.DMA,
        ),
        **_SC_PARAMS,
    )(srcp, dstp, asrc2, adst2, m2, z16)


def _sc_l2b_body(src_h, dst_h, ee_h, dinv_h, z16_h,
                 w_out,
                 sidx, didx, eebuf, bbuf, wbuf, w_sh, sem, sem2,
                 *, np_, chunks):
    c = lax.axis_index("c")
    s = lax.axis_index("s")
    wid = c * NS + s
    rows = np_ // NS
    pltpu.sync_copy(z16_h, w_sh.at[pl.ds(s * rows, rows)])
    plsc.subcore_barrier()

    base = wid * (chunks * CK)

    def chunk(ci, carry):
        off = base + ci * CK
        i1 = pltpu.async_copy(src_h.at[pl.ds(off, CK)], sidx, sem2)
        i2 = pltpu.async_copy(dst_h.at[pl.ds(off, CK)], didx, sem2)
        i3 = pltpu.async_copy(ee_h.at[pl.ds(off, CK)], eebuf, sem2)
        i1.wait()
        i2.wait()
        i3.wait()
        pltpu.async_copy(dinv_h.at[didx], bbuf, sem).wait()

        def row(i, carry2):
            wbuf[i] = eebuf[i] * bbuf[i]
            return carry2

        lax.fori_loop(0, CK, row, 0, unroll=4)

        pltpu.sync_copy(wbuf, w_sh.at[sidx], add=True)
        return carry

    lax.fori_loop(0, chunks, chunk, 0)
    plsc.subcore_barrier()
    pltpu.sync_copy(w_sh.at[pl.ds(s * rows, rows)],
                    w_out.at[c, pl.ds(s * rows, rows)])


def _sc_l2b(srcp, dstp, ee2, dinv, z16, np_, chunks):
    body = functools.partial(_sc_l2b_body, np_=np_, chunks=chunks)
    return pl.kernel(
        body,
        out_type=jax.ShapeDtypeStruct((NC, np_, 16), jnp.float32),
        mesh=plsc.VectorSubcoreMesh(**_MESH),
        scratch_types=(
            pltpu.VMEM((CK,), jnp.int32),
            pltpu.VMEM((CK,), jnp.int32),
            pltpu.VMEM((CK, 16), jnp.float32),
            pltpu.VMEM((CK, 16), jnp.float32),
            pltpu.VMEM((CK, 16), jnp.float32),
            pltpu.VMEM_SHARED((np_, 16), jnp.float32),
            pltpu.SemaphoreType.DMA,
            pltpu.SemaphoreType.DMA,
        ),
        **_SC_PARAMS,
    )(srcp, dstp, ee2, dinv, z16)


def _stack_weights(params):
    p0, p1 = params
    w1s = jnp.concatenate([p0['W1'], p1['W1']], axis=1)
    f_s = jnp.concatenate([p0['as1'].reshape(-1), p1['as1'].reshape(-1)])
    f_d = jnp.concatenate([p0['ad1'].reshape(-1), p1['ad1'].reshape(-1)])
    oh = jax.nn.one_hot(jnp.arange(128) // 8, 16, dtype=jnp.float32)
    a_s = oh * f_s[:, None]
    a_d = oh * f_d[:, None]
    b1r = jnp.broadcast_to(
        jnp.concatenate([p0['b1'], p1['b1']])[None, :], (8, 128))
    e16 = oh.T
    w2s = jnp.zeros((128, 1024), jnp.float32)
    w2s = w2s.at[:64, :512].set(p0['W2'])
    w2s = w2s.at[64:, 512:].set(p1['W2'])
    f2s = jnp.concatenate([p0['as2'].reshape(-1), p1['as2'].reshape(-1)])
    f2d = jnp.concatenate([p0['ad2'].reshape(-1), p1['ad2'].reshape(-1)])
    oh2 = jax.nn.one_hot(jnp.arange(1024) // 64, 16, dtype=jnp.float32)
    a2s = oh2 * f2s[:, None]
    a2d = oh2 * f2d[:, None]
    e2 = oh2.T
    wcb = jnp.concatenate([jnp.tile(p0['Wc'], (8, 1)),
                           jnp.tile(p1['Wc'], (8, 1))], axis=0)
    cv = 0.5 * ((p0['b2'] @ p0['Wc'] + p0['bc'])
                + (p1['b2'] @ p1['Wc'] + p1['bc']))
    cv = jnp.broadcast_to(cv[None, :], (8, 8))
    return w1s, a_s, a_d, b1r, e16, w2s, a2s, a2d, e2, wcb, cv


@jax.jit
def kernel(x, edge_index, batch, params):
    n = x.shape[0]
    e = edge_index.shape[1]
    bn = 1000
    np_ = -(-(n + 1) // 128) * 128
    chunks = -(-e // (NW * CK))
    ep = NW * CK * chunks
    src = edge_index[0]
    dst = edge_index[1]
    pad = jnp.full((ep - e,), n, jnp.int32)
    srcp = jnp.concatenate([src, pad])
    dstp = jnp.concatenate([dst, pad])
    rows = np_ // NS
    z16 = jnp.zeros((rows, 16), jnp.float32)
    z144 = jnp.zeros((rows, 144), jnp.float32)

    (w1s, a_s, a_d, b1r, e16, w2s, a2s, a2d, e2, wcb, cv) = \
        _stack_weights(params)

    comb, adst, m1 = _tc1(x, w1s, a_s, a_d, n, np_, bn)
    accp = _sc_l1(srcp, dstp, comb, adst, m1, z144, np_, chunks)
    h2, asrc2, adst2, m2 = _tc2(accp, b1r, e16, w2s, a2s, a2d, n, np_, bn)
    ee2, den2p = _sc_l2a(srcp, dstp, asrc2, adst2, m2, z16, np_, ep, chunks)
    dinv = _tc3(den2p, np_)
    wp = _sc_l2b(srcp, dstp, ee2, dinv, z16, np_, chunks)
    out = _tc4(wp, h2, e2, wcb, cv, n, bn)
    return out[0:1]
